# baseline (speedup 1.0000x reference)
import functools

import jax
import jax.numpy as jnp
from jax import lax
from jax.experimental import pallas as pl
from jax.experimental.pallas import tpu as pltpu
from jax.experimental.pallas import tpu_sc as plsc

VOCAB = 30522
HIDDEN = 1024
MAX_POS = 512
EPS = 1e-12

_INFO = plsc.get_sparse_core_info()
_NC = _INFO.num_cores
_NS = _INFO.num_subcores
_NW = _NC * _NS

_CHUNK = 32


def _make_sc_gather(total_tokens: int):
  b_per_w = total_tokens // _NW
  n_chunks = b_per_w // _CHUNK
  mesh = plsc.VectorSubcoreMesh(core_axis_name="c", subcore_axis_name="s")

  @functools.partial(
      pl.kernel,
      mesh=mesh,
      out_type=jax.ShapeDtypeStruct((total_tokens, HIDDEN), jnp.float32),
      scratch_types=[
          pltpu.VMEM((b_per_w,), jnp.int32),
          pltpu.VMEM((2, _CHUNK, HIDDEN), jnp.float32),
          pltpu.SemaphoreType.DMA,
          pltpu.SemaphoreType.DMA,
          pltpu.SemaphoreType.DMA,
          pltpu.SemaphoreType.DMA,
      ],
  )
  def sc_gather(table_hbm, idx_hbm, out_hbm, idx_v, rows_v, g0, g1, s0, s1):
    gsem = (g0, g1)
    ssem = (s0, s1)
    wid = lax.axis_index("s") * _NC + lax.axis_index("c")
    base = wid * b_per_w
    pltpu.sync_copy(idx_hbm.at[pl.ds(base, b_per_w)], idx_v)

    def gather(c):
      buf = c % 2
      return pltpu.async_copy(
          table_hbm.at[idx_v.at[pl.ds(c * _CHUNK, _CHUNK)]],
          rows_v.at[buf], gsem[buf])

    def scatter(c):
      buf = c % 2
      return pltpu.async_copy(
          rows_v.at[buf],
          out_hbm.at[pl.ds(base + c * _CHUNK, _CHUNK)], ssem[buf])

    gathers = {0: gather(0)}
    scatters = {}
    for c in range(n_chunks):
      if c + 1 < n_chunks:
        if c - 1 in scatters:
          scatters[c - 1].wait()
        gathers[c + 1] = gather(c + 1)
      gathers[c].wait()
      scatters[c] = scatter(c)
    scatters[n_chunks - 2].wait()
    scatters[n_chunks - 1].wait()

  return sc_gather


def _ln_body(words_ref, pos_ref, type_ref, tt_ref, w_ref, b_ref, out_ref):
  x = words_ref[0]
  x = x + pos_ref[...]
  ttf = tt_ref[0]
  t0 = type_ref[0, :]
  t1 = type_ref[1, :]
  x = x + t0[None, :] + ttf * (t1 - t0)[None, :]
  u = jnp.mean(x, axis=-1, keepdims=True)
  xc = x - u
  s = jnp.mean(xc * xc, axis=-1, keepdims=True)
  y = xc * lax.rsqrt(s + EPS)
  out_ref[0] = y * w_ref[0][None, :] + b_ref[0][None, :]


_NSEG = 2
_SEQ_BLK = 512


def _ln_body_carry(words_ref, pos_ref, type_ref, tt_ref, w_ref, b_ref,
                   carry_ref, out_ref):
  del carry_ref
  _ln_body(words_ref, pos_ref, type_ref, tt_ref, w_ref, b_ref, out_ref)


def _make_tc_ln_seg(batch: int, seq: int, seg_rows: int, base: int,
                    aliased: bool):
  sb = _SEQ_BLK
  common_in = [
      pl.BlockSpec((1, sb, HIDDEN), lambda i, j: (i, j, 0)),
      pl.BlockSpec((sb, HIDDEN), lambda i, j: (j, 0)),
      pl.BlockSpec((2, HIDDEN), lambda i, j: (0, 0)),
      pl.BlockSpec((1, sb, 1), lambda i, j: (i, j, 0)),
      pl.BlockSpec((1, HIDDEN), lambda i, j: (0, 0)),
      pl.BlockSpec((1, HIDDEN), lambda i, j: (0, 0)),
  ]
  if aliased:
    common_in.append(pl.BlockSpec(memory_space=pl.ANY))
  return pl.pallas_call(
      _ln_body_carry if aliased else _ln_body,
      grid=(seg_rows, seq // sb),
      in_specs=common_in,
      out_specs=pl.BlockSpec((1, sb, HIDDEN), lambda i, j: (i + base, j, 0)),
      out_shape=jax.ShapeDtypeStruct((batch, seq, HIDDEN), jnp.float32),
      input_output_aliases={6: 0} if aliased else {},
      compiler_params=pltpu.CompilerParams(
          dimension_semantics=("parallel", "parallel")),
  )


def kernel(input_ids, token_type_ids, word_emb, pos_emb, type_emb,
           ln_weight, ln_bias):
  batch, seq = input_ids.shape
  total = batch * seq
  ids_flat = input_ids.reshape(total).astype(jnp.int32)
  ttf = token_type_ids.reshape(batch, seq, 1).astype(jnp.float32)
  lnw = ln_weight.reshape(1, HIDDEN)
  lnb = ln_bias.reshape(1, HIDDEN)

  seg_tokens = total // _NSEG
  seg_rows = batch // _NSEG
  sc_gather = _make_sc_gather(seg_tokens)

  out = None
  for g in range(_NSEG):
    ids_g = lax.slice(ids_flat, (g * seg_tokens,), ((g + 1) * seg_tokens,))
    words_g = sc_gather(word_emb, ids_g).reshape(seg_rows, seq, HIDDEN)
    tt_g = lax.slice(ttf, (g * seg_rows, 0, 0),
                     ((g + 1) * seg_rows, seq, 1))
    tc_ln = _make_tc_ln_seg(batch, seq, seg_rows, g * seg_rows, g > 0)
    args = (words_g, pos_emb, type_emb, tt_g, lnw, lnb)
    out = tc_ln(*args) if g == 0 else tc_ln(*args, out)
  return out


# speedup vs baseline: 1.0112x; 1.0112x over previous
import functools

import jax
import jax.numpy as jnp
from jax import lax
from jax.experimental import pallas as pl
from jax.experimental.pallas import tpu as pltpu
from jax.experimental.pallas import tpu_sc as plsc

VOCAB = 30522
HIDDEN = 1024
MAX_POS = 512
EPS = 1e-12

_INFO = plsc.get_sparse_core_info()
_NC = _INFO.num_cores
_NS = _INFO.num_subcores
_NW = _NC * _NS

_CHUNK = 32


def _make_sc_gather(total_tokens: int):
  b_per_w = total_tokens // _NW
  n_chunks = b_per_w // _CHUNK
  mesh = plsc.VectorSubcoreMesh(core_axis_name="c", subcore_axis_name="s")

  @functools.partial(
      pl.kernel,
      mesh=mesh,
      out_type=jax.ShapeDtypeStruct((total_tokens, HIDDEN), jnp.float32),
      scratch_types=[
          pltpu.VMEM((b_per_w,), jnp.int32),
          pltpu.VMEM((2, _CHUNK, HIDDEN), jnp.float32),
          pltpu.SemaphoreType.DMA,
          pltpu.SemaphoreType.DMA,
          pltpu.SemaphoreType.DMA,
          pltpu.SemaphoreType.DMA,
      ],
  )
  def sc_gather(table_hbm, idx_hbm, out_hbm, idx_v, rows_v, g0, g1, s0, s1):
    gsem = (g0, g1)
    ssem = (s0, s1)
    wid = lax.axis_index("s") * _NC + lax.axis_index("c")
    base = wid * b_per_w
    pltpu.sync_copy(idx_hbm.at[pl.ds(base, b_per_w)], idx_v)

    def gather(c):
      buf = c % 2
      return pltpu.async_copy(
          table_hbm.at[idx_v.at[pl.ds(c * _CHUNK, _CHUNK)]],
          rows_v.at[buf], gsem[buf])

    def scatter(c):
      buf = c % 2
      return pltpu.async_copy(
          rows_v.at[buf],
          out_hbm.at[pl.ds(base + c * _CHUNK, _CHUNK)], ssem[buf])

    gathers = {0: gather(0)}
    scatters = {}
    for c in range(n_chunks):
      if c + 1 < n_chunks:
        if c - 1 in scatters:
          scatters[c - 1].wait()
        gathers[c + 1] = gather(c + 1)
      gathers[c].wait()
      scatters[c] = scatter(c)
    scatters[n_chunks - 2].wait()
    scatters[n_chunks - 1].wait()

  return sc_gather


def _ln_body(words_ref, pos_ref, type_ref, tt_ref, w_ref, b_ref, out_ref):
  x = words_ref[0]
  x = x + pos_ref[...]
  ttf = tt_ref[0]
  t0 = type_ref[0, :]
  t1 = type_ref[1, :]
  x = x + t0[None, :] + ttf * (t1 - t0)[None, :]
  u = jnp.mean(x, axis=-1, keepdims=True)
  xc = x - u
  s = jnp.mean(xc * xc, axis=-1, keepdims=True)
  y = xc * lax.rsqrt(s + EPS)
  out_ref[0] = y * w_ref[0][None, :] + b_ref[0][None, :]


_NSEG = 2
_SEQ_BLK = 512


def _ln_body_carry(words_ref, pos_ref, type_ref, tt_ref, w_ref, b_ref,
                   carry_ref, out_ref):
  del carry_ref
  _ln_body(words_ref, pos_ref, type_ref, tt_ref, w_ref, b_ref, out_ref)


def _make_tc_ln_seg(batch: int, seq: int, seg_rows: int, base: int,
                    aliased: bool):
  sb = _SEQ_BLK
  common_in = [
      pl.BlockSpec((1, sb, HIDDEN), lambda i, j: (i, j, 0)),
      pl.BlockSpec((sb, HIDDEN), lambda i, j: (j, 0)),
      pl.BlockSpec((2, HIDDEN), lambda i, j: (0, 0)),
      pl.BlockSpec((1, sb, 1), lambda i, j: (i, j, 0)),
      pl.BlockSpec((1, HIDDEN), lambda i, j: (0, 0)),
      pl.BlockSpec((1, HIDDEN), lambda i, j: (0, 0)),
  ]
  if aliased:
    common_in.append(pl.BlockSpec(memory_space=pl.ANY))
  return pl.pallas_call(
      _ln_body_carry if aliased else _ln_body,
      grid=(seg_rows, seq // sb),
      in_specs=common_in,
      out_specs=pl.BlockSpec((1, sb, HIDDEN), lambda i, j: (i + base, j, 0)),
      out_shape=jax.ShapeDtypeStruct((batch, seq, HIDDEN), jnp.float32),
      input_output_aliases={6: 0} if aliased else {},
  )


def kernel(input_ids, token_type_ids, word_emb, pos_emb, type_emb,
           ln_weight, ln_bias):
  batch, seq = input_ids.shape
  total = batch * seq
  ids_flat = input_ids.reshape(total).astype(jnp.int32)
  ttf = token_type_ids.reshape(batch, seq, 1).astype(jnp.float32)
  lnw = ln_weight.reshape(1, HIDDEN)
  lnb = ln_bias.reshape(1, HIDDEN)

  seg_tokens = total // _NSEG
  seg_rows = batch // _NSEG
  sc_gather = _make_sc_gather(seg_tokens)

  out = None
  for g in range(_NSEG):
    ids_g = lax.slice(ids_flat, (g * seg_tokens,), ((g + 1) * seg_tokens,))
    words_g = sc_gather(word_emb, ids_g).reshape(seg_rows, seq, HIDDEN)
    tt_g = lax.slice(ttf, (g * seg_rows, 0, 0),
                     ((g + 1) * seg_rows, seq, 1))
    tc_ln = _make_tc_ln_seg(batch, seq, seg_rows, g * seg_rows, g > 0)
    args = (words_g, pos_emb, type_emb, tt_g, lnw, lnb)
    out = tc_ln(*args) if g == 0 else tc_ln(*args, out)
  return out
